# baseline (speedup 1.0000x reference)
import functools

import jax
import jax.numpy as jnp
from jax import lax
from jax.experimental import pallas as pl
from jax.experimental.pallas import tpu as pltpu
from jax.experimental.pallas import tpu_sc as plsc

B = 16384
N_NUM_FEATS = 64
K_FIELD = 16
K_ID = 32
OUT_DIM = N_NUM_FEATS + 2 * K_FIELD + 3 * K_ID
N_DR = 1000
N_FIELD = 1000
N_JOCKEY = 100000
N_HORSE = 1000000
N_TRAINER = 100000

_info = plsc.get_sparse_core_info()
NC, NS, L = _info.num_cores, _info.num_subcores, _info.num_lanes
NW = NC * NS
BPW = B // NW


def _score_body(t_ref, w_ref, o_ref):
    o_ref[...] = jnp.sum(jnp.maximum(t_ref[...], 0.0) * w_ref[...], axis=0)


def _scores(tt, w2, woff, bn):
    k, n = tt.shape
    grid = (n + bn - 1) // bn
    wblk = woff // k
    return pl.pallas_call(
        _score_body,
        grid=(grid,),
        in_specs=[pl.BlockSpec((k, bn), lambda i: (0, i)),
                  pl.BlockSpec((k, 1), lambda i: (wblk, 0))],
        out_specs=pl.BlockSpec((bn,), lambda i: (i,)),
        out_shape=jax.ShapeDtypeStruct((n,), jnp.float32),
    )(tt, w2)


def _score2_body(ta_ref, tb_ref, wa_ref, wb_ref, oa_ref, ob_ref):
    oa_ref[...] = jnp.sum(jnp.maximum(ta_ref[...], 0.0) * wa_ref[...], axis=0)
    ob_ref[...] = jnp.sum(jnp.maximum(tb_ref[...], 0.0) * wb_ref[...], axis=0)


def _scores2(ta, tb, w2, woffa, woffb, bn):
    k, n = ta.shape
    grid = (n + bn - 1) // bn
    wba, wbb = woffa // k, woffb // k
    return pl.pallas_call(
        _score2_body,
        grid=(grid,),
        in_specs=[pl.BlockSpec((k, bn), lambda i: (0, i)),
                  pl.BlockSpec((k, bn), lambda i: (0, i)),
                  pl.BlockSpec((k, 1), lambda i: (wba, 0)),
                  pl.BlockSpec((k, 1), lambda i: (wbb, 0))],
        out_specs=[pl.BlockSpec((bn,), lambda i: (i,)),
                   pl.BlockSpec((bn,), lambda i: (i,))],
        out_shape=[jax.ShapeDtypeStruct((n,), jnp.float32),
                   jax.ShapeDtypeStruct((n,), jnp.float32)],
    )(ta, tb, w2, w2)


def _sc_kernel(sx_h, dr_h, field_h, jockey_h, horse_h, trainer_h,
               sd_h, sf_h, sj_h, sh_h, st_h, b_h, out_h,
               sx_v, dri_v, fi_v, ji_v, hi_v, ti_v,
               gd_v, gf_v, gj_v, gh_v, gt_v, b_v, out_v, sem):
    wid = lax.axis_index("s") * NC + lax.axis_index("c")
    base = wid * BPW

    icps = [
        pltpu.async_copy(dr_h.at[pl.ds(base, BPW)], dri_v, sem),
        pltpu.async_copy(field_h.at[pl.ds(base, BPW)], fi_v, sem),
        pltpu.async_copy(jockey_h.at[pl.ds(base, BPW)], ji_v, sem),
        pltpu.async_copy(horse_h.at[pl.ds(base, BPW)], hi_v, sem),
        pltpu.async_copy(trainer_h.at[pl.ds(base, BPW)], ti_v, sem),
        pltpu.async_copy(b_h, b_v, sem),
        pltpu.async_copy(sx_h.at[pl.ds(base, BPW)], sx_v, sem),
    ]
    for cp in icps:
        cp.wait()
    cps = [
        pltpu.async_copy(sd_h.at[dri_v], gd_v, sem),
        pltpu.async_copy(sf_h.at[fi_v], gf_v, sem),
        pltpu.async_copy(sj_h.at[ji_v], gj_v, sem),
        pltpu.async_copy(sh_h.at[hi_v], gh_v, sem),
        pltpu.async_copy(st_h.at[ti_v], gt_v, sem),
    ]
    for cp in cps:
        cp.wait()

    bias = b_v[...]

    def body(c, carry):
        sl = pl.ds(c * L, L)
        z = (sx_v[sl] + gd_v[sl] + gf_v[sl] + gj_v[sl] + gh_v[sl] + gt_v[sl]
             + bias)
        out_v[sl] = 1.0 / (1.0 + jnp.exp(-z))
        return carry

    lax.fori_loop(0, BPW // L, body, 0)
    pltpu.sync_copy(out_v, out_h.at[pl.ds(base, BPW)])


@jax.jit
def _run(x, dr, field, jockey, horse, trainer,
         emb_dr_w, emb_field_w, emb_jockey_w, emb_horse_w, emb_trainer_w,
         W, b):
    w2 = W.reshape(OUT_DIM, 1).astype(jnp.float32)

    sx = _scores(x.astype(jnp.float32).T, w2, 0, 16384)
    sd, sf = _scores2(emb_dr_w.T, emb_field_w.T, w2, 64, 80, 1024)
    sj, st = _scores2(emb_jockey_w.T, emb_trainer_w.T, w2, 96, 160, 32768)
    sh = _scores(emb_horse_w.T, w2, 128, 131072)

    b16 = jnp.broadcast_to(b.reshape(1), (L,)).astype(jnp.float32)
    mesh = plsc.VectorSubcoreMesh(core_axis_name="c", subcore_axis_name="s")
    f = functools.partial(
        pl.kernel, _sc_kernel, mesh=mesh,
        compiler_params=pltpu.CompilerParams(
            disable_bounds_checks=True, disable_semaphore_checks=True),
        out_type=jax.ShapeDtypeStruct((B,), jnp.float32),
        scratch_types=[
            pltpu.VMEM((BPW,), jnp.float32),
            pltpu.VMEM((BPW,), jnp.int32),
            pltpu.VMEM((BPW,), jnp.int32),
            pltpu.VMEM((BPW,), jnp.int32),
            pltpu.VMEM((BPW,), jnp.int32),
            pltpu.VMEM((BPW,), jnp.int32),
            pltpu.VMEM((BPW,), jnp.float32),
            pltpu.VMEM((BPW,), jnp.float32),
            pltpu.VMEM((BPW,), jnp.float32),
            pltpu.VMEM((BPW,), jnp.float32),
            pltpu.VMEM((BPW,), jnp.float32),
            pltpu.VMEM((L,), jnp.float32),
            pltpu.VMEM((BPW,), jnp.float32),
            pltpu.SemaphoreType.DMA,
        ],
    )()
    out = f(sx,
            dr.astype(jnp.int32), field.astype(jnp.int32),
            jockey.astype(jnp.int32), horse.astype(jnp.int32),
            trainer.astype(jnp.int32),
            sd, sf, sj, sh, st, b16)
    return out.reshape(B, 1)


def kernel(x, dr, field, jockey, horse, trainer, emb_dr_w, emb_field_w,
           emb_jockey_w, emb_horse_w, emb_trainer_w, W, b):
    return _run(x, dr, field, jockey, horse, trainer, emb_dr_w, emb_field_w,
                emb_jockey_w, emb_horse_w, emb_trainer_w, W, b)


# speedup vs baseline: 1.0725x; 1.0725x over previous
import functools

import jax
import jax.numpy as jnp
from jax import lax
from jax.experimental import pallas as pl
from jax.experimental.pallas import tpu as pltpu
from jax.experimental.pallas import tpu_sc as plsc

B = 16384
N_NUM_FEATS = 64
K_FIELD = 16
K_ID = 32
OUT_DIM = N_NUM_FEATS + 2 * K_FIELD + 3 * K_ID
N_DR = 1000
N_FIELD = 1000
N_JOCKEY = 100000
N_HORSE = 1000000
N_TRAINER = 100000

_info = plsc.get_sparse_core_info()
NC, NS, L = _info.num_cores, _info.num_subcores, _info.num_lanes
NW = NC * NS
BPW = B // NW


def _score_body(t_ref, w_ref, o_ref):
    o_ref[...] = jnp.sum(jnp.maximum(t_ref[...], 0.0) * w_ref[...], axis=0)


def _scores(tt, w2, woff, bn):
    k, n = tt.shape
    grid = (n + bn - 1) // bn
    wblk = woff // k
    return pl.pallas_call(
        _score_body,
        grid=(grid,),
        in_specs=[pl.BlockSpec((k, bn), lambda i: (0, i)),
                  pl.BlockSpec((k, 1), lambda i: (wblk, 0))],
        out_specs=pl.BlockSpec((bn,), lambda i: (i,)),
        out_shape=jax.ShapeDtypeStruct((n,), jnp.float32),
    )(tt, w2)


def _score2_body(ta_ref, tb_ref, wa_ref, wb_ref, oa_ref, ob_ref):
    oa_ref[...] = jnp.sum(jnp.maximum(ta_ref[...], 0.0) * wa_ref[...], axis=0)
    ob_ref[...] = jnp.sum(jnp.maximum(tb_ref[...], 0.0) * wb_ref[...], axis=0)


def _scores2(ta, tb, w2, woffa, woffb, bn):
    k, n = ta.shape
    grid = (n + bn - 1) // bn
    wba, wbb = woffa // k, woffb // k
    return pl.pallas_call(
        _score2_body,
        grid=(grid,),
        in_specs=[pl.BlockSpec((k, bn), lambda i: (0, i)),
                  pl.BlockSpec((k, bn), lambda i: (0, i)),
                  pl.BlockSpec((k, 1), lambda i: (wba, 0)),
                  pl.BlockSpec((k, 1), lambda i: (wbb, 0))],
        out_specs=[pl.BlockSpec((bn,), lambda i: (i,)),
                   pl.BlockSpec((bn,), lambda i: (i,))],
        out_shape=[jax.ShapeDtypeStruct((n,), jnp.float32),
                   jax.ShapeDtypeStruct((n,), jnp.float32)],
    )(ta, tb, w2, w2)


def _sc_kernel(sx_h, dr_h, field_h, jockey_h, horse_h, trainer_h,
               sd_h, sf_h, sj_h, sh_h, st_h, b_h, out_h,
               sx_v, dri_v, fi_v, ji_v, hi_v, ti_v,
               gd_v, gf_v, gj_v, gh_v, gt_v, b_v, out_v, sem):
    wid = lax.axis_index("s") * NC + lax.axis_index("c")
    base = wid * BPW

    icps = [
        pltpu.async_copy(dr_h.at[pl.ds(base, BPW)], dri_v, sem),
        pltpu.async_copy(field_h.at[pl.ds(base, BPW)], fi_v, sem),
        pltpu.async_copy(jockey_h.at[pl.ds(base, BPW)], ji_v, sem),
        pltpu.async_copy(horse_h.at[pl.ds(base, BPW)], hi_v, sem),
        pltpu.async_copy(trainer_h.at[pl.ds(base, BPW)], ti_v, sem),
        pltpu.async_copy(b_h, b_v, sem),
        pltpu.async_copy(sx_h.at[pl.ds(base, BPW)], sx_v, sem),
    ]
    for cp in icps:
        cp.wait()
    cps = [
        pltpu.async_copy(sd_h.at[dri_v], gd_v, sem),
        pltpu.async_copy(sf_h.at[fi_v], gf_v, sem),
        pltpu.async_copy(sj_h.at[ji_v], gj_v, sem),
        pltpu.async_copy(sh_h.at[hi_v], gh_v, sem),
        pltpu.async_copy(st_h.at[ti_v], gt_v, sem),
    ]
    for cp in cps:
        cp.wait()

    bias = b_v[...]

    def body(c, carry):
        sl = pl.ds(c * L, L)
        z = (sx_v[sl] + gd_v[sl] + gf_v[sl] + gj_v[sl] + gh_v[sl] + gt_v[sl]
             + bias)
        out_v[sl] = 1.0 / (1.0 + jnp.exp(-z))
        return carry

    lax.fori_loop(0, BPW // L, body, 0)
    pltpu.sync_copy(out_v, out_h.at[pl.ds(base, BPW)])


@jax.jit
def _run(x, dr, field, jockey, horse, trainer,
         emb_dr_w, emb_field_w, emb_jockey_w, emb_horse_w, emb_trainer_w,
         W, b):
    w2 = W.reshape(OUT_DIM, 1).astype(jnp.float32)

    sx = _scores(x.astype(jnp.float32).T, w2, 0, 16384)
    sd, sf = _scores2(emb_dr_w.T, emb_field_w.T, w2, 64, 80, 1024)
    sj = _scores(emb_jockey_w.T, w2, 96, 65536)
    st = _scores(emb_trainer_w.T, w2, 160, 65536)
    sh = _scores(emb_horse_w.T, w2, 128, 131072)

    b16 = jnp.broadcast_to(b.reshape(1), (L,)).astype(jnp.float32)
    mesh = plsc.VectorSubcoreMesh(core_axis_name="c", subcore_axis_name="s")
    f = functools.partial(
        pl.kernel, _sc_kernel, mesh=mesh,
        compiler_params=pltpu.CompilerParams(
            disable_bounds_checks=True, disable_semaphore_checks=True),
        out_type=jax.ShapeDtypeStruct((B,), jnp.float32),
        scratch_types=[
            pltpu.VMEM((BPW,), jnp.float32),
            pltpu.VMEM((BPW,), jnp.int32),
            pltpu.VMEM((BPW,), jnp.int32),
            pltpu.VMEM((BPW,), jnp.int32),
            pltpu.VMEM((BPW,), jnp.int32),
            pltpu.VMEM((BPW,), jnp.int32),
            pltpu.VMEM((BPW,), jnp.float32),
            pltpu.VMEM((BPW,), jnp.float32),
            pltpu.VMEM((BPW,), jnp.float32),
            pltpu.VMEM((BPW,), jnp.float32),
            pltpu.VMEM((BPW,), jnp.float32),
            pltpu.VMEM((L,), jnp.float32),
            pltpu.VMEM((BPW,), jnp.float32),
            pltpu.SemaphoreType.DMA,
        ],
    )()
    out = f(sx,
            dr.astype(jnp.int32), field.astype(jnp.int32),
            jockey.astype(jnp.int32), horse.astype(jnp.int32),
            trainer.astype(jnp.int32),
            sd, sf, sj, sh, st, b16)
    return out.reshape(B, 1)


def kernel(x, dr, field, jockey, horse, trainer, emb_dr_w, emb_field_w,
           emb_jockey_w, emb_horse_w, emb_trainer_w, W, b):
    return _run(x, dr, field, jockey, horse, trainer, emb_dr_w, emb_field_w,
                emb_jockey_w, emb_horse_w, emb_trainer_w, W, b)


# speedup vs baseline: 1.0784x; 1.0055x over previous
import functools

import jax
import jax.numpy as jnp
from jax import lax
from jax.experimental import pallas as pl
from jax.experimental.pallas import tpu as pltpu
from jax.experimental.pallas import tpu_sc as plsc

B = 16384
N_NUM_FEATS = 64
K_FIELD = 16
K_ID = 32
OUT_DIM = N_NUM_FEATS + 2 * K_FIELD + 3 * K_ID
N_DR = 1000
N_FIELD = 1000
N_JOCKEY = 100000
N_HORSE = 1000000
N_TRAINER = 100000

_info = plsc.get_sparse_core_info()
NC, NS, L = _info.num_cores, _info.num_subcores, _info.num_lanes
NW = NC * NS
BPW = B // NW


def _score_body(t_ref, w_ref, o_ref):
    o_ref[...] = jnp.sum(jnp.maximum(t_ref[...], 0.0) * w_ref[...], axis=0)


def _scores(tt, w2, woff, bn):
    k, n = tt.shape
    grid = (n + bn - 1) // bn
    wblk = woff // k
    return pl.pallas_call(
        _score_body,
        grid=(grid,),
        in_specs=[pl.BlockSpec((k, bn), lambda i: (0, i)),
                  pl.BlockSpec((k, 1), lambda i: (wblk, 0))],
        out_specs=pl.BlockSpec((bn,), lambda i: (i,)),
        out_shape=jax.ShapeDtypeStruct((n,), jnp.float32),
    )(tt, w2)


def _score2_body(ta_ref, tb_ref, wa_ref, wb_ref, oa_ref, ob_ref):
    oa_ref[...] = jnp.sum(jnp.maximum(ta_ref[...], 0.0) * wa_ref[...], axis=0)
    ob_ref[...] = jnp.sum(jnp.maximum(tb_ref[...], 0.0) * wb_ref[...], axis=0)


def _scores2(ta, tb, w2, woffa, woffb, bn):
    k, n = ta.shape
    grid = (n + bn - 1) // bn
    wba, wbb = woffa // k, woffb // k
    return pl.pallas_call(
        _score2_body,
        grid=(grid,),
        in_specs=[pl.BlockSpec((k, bn), lambda i: (0, i)),
                  pl.BlockSpec((k, bn), lambda i: (0, i)),
                  pl.BlockSpec((k, 1), lambda i: (wba, 0)),
                  pl.BlockSpec((k, 1), lambda i: (wbb, 0))],
        out_specs=[pl.BlockSpec((bn,), lambda i: (i,)),
                   pl.BlockSpec((bn,), lambda i: (i,))],
        out_shape=[jax.ShapeDtypeStruct((n,), jnp.float32),
                   jax.ShapeDtypeStruct((n,), jnp.float32)],
    )(ta, tb, w2, w2)


def _sc_a_kernel(sx_h, dr_h, field_h, jockey_h, trainer_h,
                 sd_h, sf_h, sj_h, st_h, b_h, z_h,
                 sx_v, dri_v, fi_v, ji_v, ti_v,
                 gd_v, gf_v, gj_v, gt_v, b_v, z_v, sem):
    wid = lax.axis_index("s") * NC + lax.axis_index("c")
    base = wid * BPW

    icps = [
        pltpu.async_copy(dr_h.at[pl.ds(base, BPW)], dri_v, sem),
        pltpu.async_copy(field_h.at[pl.ds(base, BPW)], fi_v, sem),
        pltpu.async_copy(jockey_h.at[pl.ds(base, BPW)], ji_v, sem),
        pltpu.async_copy(trainer_h.at[pl.ds(base, BPW)], ti_v, sem),
        pltpu.async_copy(b_h, b_v, sem),
        pltpu.async_copy(sx_h.at[pl.ds(base, BPW)], sx_v, sem),
    ]
    for cp in icps:
        cp.wait()
    cps = [
        pltpu.async_copy(sd_h.at[dri_v], gd_v, sem),
        pltpu.async_copy(sf_h.at[fi_v], gf_v, sem),
        pltpu.async_copy(sj_h.at[ji_v], gj_v, sem),
        pltpu.async_copy(st_h.at[ti_v], gt_v, sem),
    ]
    for cp in cps:
        cp.wait()

    bias = b_v[...]

    def body(c, carry):
        sl = pl.ds(c * L, L)
        z_v[sl] = (sx_v[sl] + gd_v[sl] + gf_v[sl] + gj_v[sl] + gt_v[sl]
                   + bias)
        return carry

    lax.fori_loop(0, BPW // L, body, 0)
    pltpu.sync_copy(z_v, z_h.at[pl.ds(base, BPW)])


def _sc_b_kernel(z_h, horse_h, sh_h, out_h,
                 z_v, hi_v, gh_v, out_v, sem):
    wid = lax.axis_index("s") * NC + lax.axis_index("c")
    base = wid * BPW

    icps = [
        pltpu.async_copy(horse_h.at[pl.ds(base, BPW)], hi_v, sem),
        pltpu.async_copy(z_h.at[pl.ds(base, BPW)], z_v, sem),
    ]
    for cp in icps:
        cp.wait()
    pltpu.async_copy(sh_h.at[hi_v], gh_v, sem).wait()

    def body(c, carry):
        sl = pl.ds(c * L, L)
        z = z_v[sl] + gh_v[sl]
        out_v[sl] = 1.0 / (1.0 + jnp.exp(-z))
        return carry

    lax.fori_loop(0, BPW // L, body, 0)
    pltpu.sync_copy(out_v, out_h.at[pl.ds(base, BPW)])


@jax.jit
def _run(x, dr, field, jockey, horse, trainer,
         emb_dr_w, emb_field_w, emb_jockey_w, emb_horse_w, emb_trainer_w,
         W, b):
    w2 = W.reshape(OUT_DIM, 1).astype(jnp.float32)

    sx = _scores(x.astype(jnp.float32).T, w2, 0, 16384)
    sd, sf = _scores2(emb_dr_w.T, emb_field_w.T, w2, 64, 80, 1024)
    sj = _scores(emb_jockey_w.T, w2, 96, 65536)
    st = _scores(emb_trainer_w.T, w2, 160, 65536)
    sh = _scores(emb_horse_w.T, w2, 128, 131072)

    b16 = jnp.broadcast_to(b.reshape(1), (L,)).astype(jnp.float32)
    mesh = plsc.VectorSubcoreMesh(core_axis_name="c", subcore_axis_name="s")
    scp = pltpu.CompilerParams(
        disable_bounds_checks=True, disable_semaphore_checks=True)
    fa = functools.partial(
        pl.kernel, _sc_a_kernel, mesh=mesh, compiler_params=scp,
        out_type=jax.ShapeDtypeStruct((B,), jnp.float32),
        scratch_types=[
            pltpu.VMEM((BPW,), jnp.float32),
            pltpu.VMEM((BPW,), jnp.int32),
            pltpu.VMEM((BPW,), jnp.int32),
            pltpu.VMEM((BPW,), jnp.int32),
            pltpu.VMEM((BPW,), jnp.int32),
            pltpu.VMEM((BPW,), jnp.float32),
            pltpu.VMEM((BPW,), jnp.float32),
            pltpu.VMEM((BPW,), jnp.float32),
            pltpu.VMEM((BPW,), jnp.float32),
            pltpu.VMEM((L,), jnp.float32),
            pltpu.VMEM((BPW,), jnp.float32),
            pltpu.SemaphoreType.DMA,
        ],
    )()
    z0 = fa(sx,
            dr.astype(jnp.int32), field.astype(jnp.int32),
            jockey.astype(jnp.int32), trainer.astype(jnp.int32),
            sd, sf, sj, st, b16)
    fb = functools.partial(
        pl.kernel, _sc_b_kernel, mesh=mesh, compiler_params=scp,
        out_type=jax.ShapeDtypeStruct((B,), jnp.float32),
        scratch_types=[
            pltpu.VMEM((BPW,), jnp.float32),
            pltpu.VMEM((BPW,), jnp.int32),
            pltpu.VMEM((BPW,), jnp.float32),
            pltpu.VMEM((BPW,), jnp.float32),
            pltpu.SemaphoreType.DMA,
        ],
    )()
    out = fb(z0, horse.astype(jnp.int32), sh)
    return out.reshape(B, 1)


def kernel(x, dr, field, jockey, horse, trainer, emb_dr_w, emb_field_w,
           emb_jockey_w, emb_horse_w, emb_trainer_w, W, b):
    return _run(x, dr, field, jockey, horse, trainer, emb_dr_w, emb_field_w,
                emb_jockey_w, emb_horse_w, emb_trainer_w, W, b)
